# baseline (speedup 1.0000x reference)
import jax
import jax.numpy as jnp
from jax import lax
from jax.experimental import pallas as pl
from jax.experimental.pallas import tpu as pltpu
from jax.experimental.pallas import tpu_sc as plsc

B = 4
IN_SZ = 40962
OUT_SZ = 163842
D = 128
NEW = OUT_SZ - IN_SZ
NC, NS = 2, 16
NW = NC * NS

K = 128
GPW = NEW // NW
GCH = GPW // K
T = GCH * B
IPW = 1280
TAIL = IN_SZ - NW * IPW

_mesh = plsc.VectorSubcoreMesh(
    core_axis_name="c", subcore_axis_name="s", num_cores=NC, num_subcores=NS)

_SCRATCH = [
    pltpu.VMEM((T * K,), jnp.int32),
    pltpu.VMEM((T * K,), jnp.int32),
    pltpu.VMEM((K, D), jnp.float32),
    pltpu.VMEM((K, D), jnp.float32),
    pltpu.VMEM((K, D), jnp.float32),
    pltpu.VMEM((K, D), jnp.float32),
    pltpu.SemaphoreType.DMA,
    pltpu.SemaphoreType.DMA,
    pltpu.SemaphoreType.DMA,
    pltpu.SemaphoreType.DMA,
    pltpu.SemaphoreType.DMA,
    pltpu.SemaphoreType.DMA,
]


def _upsample_body(x_hbm, li_hbm, ri_hbm, out_hbm,
                   idx_l, idx_r, rl0, rr0, rl1, rr1,
                   s_idx, s_g0, s_g1, s_o0, s_o1, s_id):
    wid = lax.axis_index("s") * NC + lax.axis_index("c")
    rls = (rl0, rl1)
    rrs = (rr0, rr1)
    gsems = (s_g0, s_g1)
    osems = (s_o0, s_o1)

    for b in range(B):
        pltpu.async_copy(x_hbm.at[pl.ds(b * IN_SZ + wid * IPW, IPW)],
                         out_hbm.at[pl.ds(b * OUT_SZ + wid * IPW, IPW)], s_id)

    @pl.when(wid == 0)
    def _tail():
        for b in range(B):
            pltpu.async_copy(x_hbm.at[pl.ds(b * IN_SZ + NW * IPW, TAIL)],
                             out_hbm.at[pl.ds(b * OUT_SZ + NW * IPW, TAIL)],
                             s_id)

    cl = pltpu.async_copy(li_hbm.at[pl.ds(wid * T * K, T * K)], idx_l, s_idx)
    cr = pltpu.async_copy(ri_hbm.at[pl.ds(wid * T * K, T * K)], idx_r, s_idx)
    cl.wait()
    cr.wait()

    def issue_gather(t, p):
        off = t * K
        pltpu.async_copy(x_hbm.at[idx_l.at[pl.ds(off, K)]], rls[p], gsems[p])
        pltpu.async_copy(x_hbm.at[idx_r.at[pl.ds(off, K)]], rrs[p], gsems[p])

    def wait_gather(p):
        pltpu.make_async_copy(x_hbm.at[pl.ds(0, K)], rls[p], gsems[p]).wait()
        pltpu.make_async_copy(x_hbm.at[pl.ds(0, K)], rrs[p], gsems[p]).wait()

    def wait_out(p):
        pltpu.make_async_copy(x_hbm.at[pl.ds(0, K)], rls[p], osems[p]).wait()

    def issue_out(t, p):
        b = lax.bitwise_and(t, B - 1)
        c = lax.shift_right_logical(t, 2)
        row0 = b * OUT_SZ + IN_SZ + wid * GPW + c * K
        pltpu.async_copy(rls[p], out_hbm.at[pl.ds(row0, K)], osems[p])

    def avg(p):
        rl, rr = rls[p], rrs[p]

        def row(i, carry):
            for j in range(D // 16):
                s = pl.ds(j * 16, 16)
                rl[i, s] = (rl[i, s] + rr[i, s]) * 0.5
            return carry

        lax.fori_loop(0, K, row, 0)

    def step(t, p, first, last):
        q = 1 - p
        if first is None:
            wait_out(q)
        else:
            pl.when(jnp.logical_not(first))(lambda: wait_out(q))
        if last is None:
            issue_gather(t + 1, q)
        else:
            pl.when(jnp.logical_not(last))(lambda: issue_gather(t + 1, q))
        wait_gather(p)
        avg(p)
        issue_out(t, p)

    issue_gather(0, 0)

    def two_steps(k, carry):
        t0 = 2 * k
        step(t0, 0, first=(k == 0), last=None)
        step(t0 + 1, 1, first=None, last=(k == T // 2 - 1))
        return carry

    lax.fori_loop(0, T // 2, two_steps, 0)

    wait_out(1)
    for b in range(B):
        pltpu.make_async_copy(
            x_hbm.at[pl.ds(b * IN_SZ + wid * IPW, IPW)],
            out_hbm.at[pl.ds(b * OUT_SZ + wid * IPW, IPW)], s_id).wait()

    @pl.when(wid == 0)
    def _tail_drain():
        for b in range(B):
            pltpu.make_async_copy(
                x_hbm.at[pl.ds(b * IN_SZ + NW * IPW, TAIL)],
                out_hbm.at[pl.ds(b * OUT_SZ + NW * IPW, TAIL)], s_id).wait()


_upsample = pl.kernel(
    _upsample_body,
    out_type=jax.ShapeDtypeStruct((B * OUT_SZ, D), jnp.float32),
    mesh=_mesh,
    compiler_params=pltpu.CompilerParams(use_tc_tiling_on_sc=False),
    scratch_types=_SCRATCH,
)


def _expand_indices(idx):
    i = idx.astype(jnp.int32).reshape(NW, GCH, 1, K)
    shift = (jnp.arange(B, dtype=jnp.int32) * IN_SZ).reshape(1, 1, B, 1)
    return (i + shift).reshape(NW * T * K)


def kernel(x, left_idx, right_idx):
    x_flat = x.reshape(B * IN_SZ, D)
    li = _expand_indices(left_idx[IN_SZ:])
    ri = _expand_indices(right_idx[IN_SZ:])
    out = _upsample(x_flat, li, ri)
    return out.reshape(B, OUT_SZ, D)


# speedup vs baseline: 2.2061x; 2.2061x over previous
import jax
import jax.numpy as jnp
from jax import lax
from jax.experimental import pallas as pl
from jax.experimental.pallas import tpu as pltpu
from jax.experimental.pallas import tpu_sc as plsc

B = 4
IN_SZ = 40962
OUT_SZ = 163842
D = 128
W = B * D
NEW = OUT_SZ - IN_SZ
NC, NS = 2, 16
NW = NC * NS

GPW = NEW // NW
K = 48
T = GPW // K

IPW = 1280
IK = 64
TAIL = IN_SZ - NW * IPW

VK = 64
VCH = IPW // VK

_mesh = plsc.VectorSubcoreMesh(
    core_axis_name="c", subcore_axis_name="s", num_cores=NC, num_subcores=NS)
_params = pltpu.CompilerParams(use_tc_tiling_on_sc=False)


def _repack_body(x_hbm, xv_hbm, buf0, buf1, s_in0, s_in1, s_out0, s_out1):
    wid = lax.axis_index("s") * NC + lax.axis_index("c")
    bufs = (buf0, buf1)
    isems = (s_in0, s_in1)
    osems = (s_out0, s_out1)
    base = wid * IPW

    def issue_in(t, p):
        v0 = base + t * VK
        for b in range(B):
            pltpu.async_copy(x_hbm.at[pl.ds(b * IN_SZ + v0, VK)],
                             bufs[p].at[:, pl.ds(b * D, D)], isems[p])

    def wait_in(p):
        for b in range(B):
            pltpu.make_async_copy(x_hbm.at[pl.ds(0, VK)],
                                  bufs[p].at[:, pl.ds(b * D, D)],
                                  isems[p]).wait()

    def issue_out(t, p):
        pltpu.async_copy(bufs[p], xv_hbm.at[pl.ds(base + t * VK, VK)],
                         osems[p])

    def wait_out(p):
        pltpu.make_async_copy(x_hbm.at[pl.ds(0, VK)], bufs[p],
                              osems[p]).wait()

    def step(t, p, first=False, last=None):
        q = 1 - p
        if not first:
            wait_out(q)
        if last is None:
            issue_in(t + 1, q)
        else:
            pl.when(jnp.logical_not(last))(lambda: issue_in(t + 1, q))
        wait_in(p)
        issue_out(t, p)

    issue_in(0, 0)
    step(0, 0, first=True)
    step(1, 1)

    def two_steps(k, carry):
        t0 = 2 * k
        step(t0, 0)
        step(t0 + 1, 1, last=(k == VCH // 2 - 1))
        return carry

    lax.fori_loop(1, VCH // 2, two_steps, 0)

    wait_out(1)

    @pl.when(wid == 0)
    def _tail():
        v0 = NW * IPW
        for b in range(B):
            pltpu.sync_copy(x_hbm.at[pl.ds(b * IN_SZ + v0, TAIL)],
                            buf0.at[pl.ds(0, TAIL), pl.ds(b * D, D)])
        pltpu.sync_copy(buf0.at[pl.ds(0, TAIL)], xv_hbm.at[pl.ds(v0, TAIL)])


_repack = pl.kernel(
    _repack_body,
    out_type=jax.ShapeDtypeStruct((IN_SZ, W), jnp.float32),
    mesh=_mesh,
    compiler_params=_params,
    scratch_types=[
        pltpu.VMEM((VK, W), jnp.float32),
        pltpu.VMEM((VK, W), jnp.float32),
        pltpu.SemaphoreType.DMA,
        pltpu.SemaphoreType.DMA,
        pltpu.SemaphoreType.DMA,
        pltpu.SemaphoreType.DMA,
    ],
)


def _upsample_body(xv_hbm, x_hbm, li_hbm, ri_hbm, out_hbm,
                   idx_l, idx_r, rl0, rr0, rl1, rr1, id0, id1,
                   s_g0, s_g1, s_o0, s_o1, s_ii0, s_ii1, s_io0, s_io1):
    wid = lax.axis_index("s") * NC + lax.axis_index("c")
    rls = (rl0, rl1)
    rrs = (rr0, rr1)
    ids = (id0, id1)
    gsems = (s_g0, s_g1)
    osems = (s_o0, s_o1)
    iisems = (s_ii0, s_ii1)
    iosems = (s_io0, s_io1)

    cl = pltpu.async_copy(li_hbm.at[pl.ds(wid * GPW, GPW)], idx_l, s_g0)
    cr = pltpu.async_copy(ri_hbm.at[pl.ds(wid * GPW, GPW)], idx_r, s_g1)
    cl.wait()
    cr.wait()

    def issue_gather(t, p):
        off = t * K
        pltpu.async_copy(xv_hbm.at[idx_l.at[pl.ds(off, K)]], rls[p], gsems[p])
        pltpu.async_copy(xv_hbm.at[idx_r.at[pl.ds(off, K)]], rrs[p], gsems[p])

    def wait_gather(p):
        pltpu.make_async_copy(xv_hbm.at[pl.ds(0, K)], rls[p], gsems[p]).wait()
        pltpu.make_async_copy(xv_hbm.at[pl.ds(0, K)], rrs[p], gsems[p]).wait()

    def issue_out(t, p):
        v0 = IN_SZ + wid * GPW + t * K
        for b in range(B):
            pltpu.async_copy(rls[p].at[:, pl.ds(b * D, D)],
                             out_hbm.at[pl.ds(b * OUT_SZ + v0, K)], osems[p])

    def wait_out(p):
        for b in range(B):
            pltpu.make_async_copy(xv_hbm.at[pl.ds(0, K)],
                                  rls[p].at[:, pl.ds(b * D, D)],
                                  osems[p]).wait()

    def avg(p):
        rl, rr = rls[p], rrs[p]

        def row(i, carry):
            for j in range(W // 16):
                s = pl.ds(j * 16, 16)
                rl[i, s] = (rl[i, s] + rr[i, s]) * 0.5
            return carry

        lax.fori_loop(0, K, row, 0)

    def id_rows(t):
        b = lax.bitwise_and(t, B - 1)
        j = lax.shift_right_logical(t, 2)
        return b, wid * IPW + j * IK

    def issue_id_in(t, p):
        b, r0 = id_rows(t)
        pltpu.async_copy(x_hbm.at[pl.ds(b * IN_SZ + r0, IK)], ids[p],
                         iisems[p])

    def wait_id_in(p):
        pltpu.make_async_copy(x_hbm.at[pl.ds(0, IK)], ids[p],
                              iisems[p]).wait()

    def issue_id_out(t, p):
        b, r0 = id_rows(t)
        pltpu.async_copy(ids[p], out_hbm.at[pl.ds(b * OUT_SZ + r0, IK)],
                         iosems[p])

    def wait_id_out(p):
        pltpu.make_async_copy(x_hbm.at[pl.ds(0, IK)], ids[p],
                              iosems[p]).wait()

    def step(t, p, first=False, last=None):
        q = 1 - p
        if not first:
            wait_id_out(q)
            wait_out(q)
        if last is None:
            issue_id_in(t + 1, q)
            issue_gather(t + 1, q)
        else:
            def _issue_next():
                issue_id_in(t + 1, q)
                issue_gather(t + 1, q)

            pl.when(jnp.logical_not(last))(_issue_next)
        wait_gather(p)
        avg(p)
        issue_out(t, p)
        wait_id_in(p)
        issue_id_out(t, p)

    issue_gather(0, 0)
    issue_id_in(0, 0)
    step(0, 0, first=True)
    step(1, 1)

    def two_steps(k, carry):
        t0 = 2 * k
        step(t0, 0)
        step(t0 + 1, 1, last=(k == T // 2 - 1))
        return carry

    lax.fori_loop(1, T // 2, two_steps, 0)

    wait_out(1)
    wait_id_out(1)

    @pl.when(wid == 0)
    def _tail():
        r0 = NW * IPW
        for b in range(B):
            pltpu.sync_copy(x_hbm.at[pl.ds(b * IN_SZ + r0, TAIL)],
                            id0.at[pl.ds(0, TAIL)])
            pltpu.sync_copy(id0.at[pl.ds(0, TAIL)],
                            out_hbm.at[pl.ds(b * OUT_SZ + r0, TAIL)])


_upsample = pl.kernel(
    _upsample_body,
    out_type=jax.ShapeDtypeStruct((B * OUT_SZ, D), jnp.float32),
    mesh=_mesh,
    compiler_params=_params,
    scratch_types=[
        pltpu.VMEM((GPW,), jnp.int32),
        pltpu.VMEM((GPW,), jnp.int32),
        pltpu.VMEM((K, W), jnp.float32),
        pltpu.VMEM((K, W), jnp.float32),
        pltpu.VMEM((K, W), jnp.float32),
        pltpu.VMEM((K, W), jnp.float32),
        pltpu.VMEM((IK, D), jnp.float32),
        pltpu.VMEM((IK, D), jnp.float32),
        pltpu.SemaphoreType.DMA,
        pltpu.SemaphoreType.DMA,
        pltpu.SemaphoreType.DMA,
        pltpu.SemaphoreType.DMA,
        pltpu.SemaphoreType.DMA,
        pltpu.SemaphoreType.DMA,
        pltpu.SemaphoreType.DMA,
        pltpu.SemaphoreType.DMA,
    ],
)


def kernel(x, left_idx, right_idx):
    x_flat = x.reshape(B * IN_SZ, D)
    li = left_idx[IN_SZ:].astype(jnp.int32)
    ri = right_idx[IN_SZ:].astype(jnp.int32)
    xv = _repack(x_flat)
    out = _upsample(xv, x_flat, li, ri)
    return out.reshape(B, OUT_SZ, D)


# speedup vs baseline: 10.4503x; 4.7369x over previous
import jax
import jax.numpy as jnp
from jax import lax
from jax.experimental import pallas as pl
from jax.experimental.pallas import tpu as pltpu
from jax.experimental.pallas import tpu_sc as plsc

B = 4
IN_SZ = 40962
OUT_SZ = 163842
D = 128
W = B * D
NC, NS = 2, 16
NW = NC * NS

RPW = 5120
K = 40
T = RPW // K
TAIL = OUT_SZ - NW * RPW

_mesh = plsc.VectorSubcoreMesh(
    core_axis_name="c", subcore_axis_name="s", num_cores=NC, num_subcores=NS)
_params = pltpu.CompilerParams(use_tc_tiling_on_sc=False)


def _upsample_body(xv_hbm, li_hbm, ri_hbm, out_hbm,
                   idx_l, idx_r, tidx_l, tidx_r, rl0, rr0, rl1, rr1,
                   s_g0, s_g1, s_o0, s_o1):
    wid = lax.axis_index("s") * NC + lax.axis_index("c")
    rls = (rl0, rl1)
    rrs = (rr0, rr1)
    gsems = (s_g0, s_g1)
    osems = (s_o0, s_o1)

    cl = pltpu.async_copy(li_hbm.at[pl.ds(wid * RPW, RPW)], idx_l, s_g0)
    cr = pltpu.async_copy(ri_hbm.at[pl.ds(wid * RPW, RPW)], idx_r, s_g1)
    cl.wait()
    cr.wait()

    def issue_gather(t, p):
        off = t * K
        pltpu.async_copy(xv_hbm.at[idx_l.at[pl.ds(off, K)]], rls[p], gsems[p])
        pltpu.async_copy(xv_hbm.at[idx_r.at[pl.ds(off, K)]], rrs[p], gsems[p])

    def wait_gather(p):
        pltpu.make_async_copy(xv_hbm.at[pl.ds(0, K)], rls[p], gsems[p]).wait()
        pltpu.make_async_copy(xv_hbm.at[pl.ds(0, K)], rrs[p], gsems[p]).wait()

    def issue_out(t, p):
        pltpu.async_copy(rls[p], out_hbm.at[pl.ds(wid * RPW + t * K, K)],
                         osems[p])

    def wait_out(p):
        pltpu.make_async_copy(xv_hbm.at[pl.ds(0, K)], rls[p], osems[p]).wait()

    def avg(p, rows=K):
        rl, rr = rls[p], rrs[p]

        def row(i, carry):
            for j in range(W // 16):
                s = pl.ds(j * 16, 16)
                rl[i, s] = (rl[i, s] + rr[i, s]) * 0.5
            return carry

        lax.fori_loop(0, rows, row, 0)

    def step(t, p, first=False, last=None):
        q = 1 - p
        if not first:
            wait_out(q)
        if last is None:
            issue_gather(t + 1, q)
        else:
            pl.when(jnp.logical_not(last))(lambda: issue_gather(t + 1, q))
        wait_gather(p)
        avg(p)
        issue_out(t, p)

    issue_gather(0, 0)
    step(0, 0, first=True)
    step(1, 1)

    def two_steps(k, carry):
        t0 = 2 * k
        step(t0, 0)
        step(t0 + 1, 1, last=(k == T // 2 - 1))
        return carry

    lax.fori_loop(1, T // 2, two_steps, 0)

    wait_out(1)

    @pl.when(wid == 0)
    def _tail():
        r0 = NW * RPW
        pltpu.sync_copy(li_hbm.at[pl.ds(r0, TAIL)], tidx_l.at[pl.ds(0, TAIL)])
        pltpu.sync_copy(ri_hbm.at[pl.ds(r0, TAIL)], tidx_r.at[pl.ds(0, TAIL)])
        cl2 = pltpu.async_copy(xv_hbm.at[tidx_l.at[pl.ds(0, TAIL)]],
                               rl0.at[pl.ds(0, TAIL)], s_g0)
        cr2 = pltpu.async_copy(xv_hbm.at[tidx_r.at[pl.ds(0, TAIL)]],
                               rr0.at[pl.ds(0, TAIL)], s_g1)
        cl2.wait()
        cr2.wait()

        def trow(i, carry):
            for j in range(W // 16):
                s = pl.ds(j * 16, 16)
                rl0[i, s] = (rl0[i, s] + rr0[i, s]) * 0.5
            return carry

        lax.fori_loop(0, TAIL, trow, 0)
        pltpu.sync_copy(rl0.at[pl.ds(0, TAIL)], out_hbm.at[pl.ds(r0, TAIL)])


_upsample = pl.kernel(
    _upsample_body,
    out_type=jax.ShapeDtypeStruct((OUT_SZ, W), jnp.float32),
    mesh=_mesh,
    compiler_params=_params,
    scratch_types=[
        pltpu.VMEM((RPW,), jnp.int32),
        pltpu.VMEM((RPW,), jnp.int32),
        pltpu.VMEM((8,), jnp.int32),
        pltpu.VMEM((8,), jnp.int32),
        pltpu.VMEM((K, W), jnp.float32),
        pltpu.VMEM((K, W), jnp.float32),
        pltpu.VMEM((K, W), jnp.float32),
        pltpu.VMEM((K, W), jnp.float32),
        pltpu.SemaphoreType.DMA,
        pltpu.SemaphoreType.DMA,
        pltpu.SemaphoreType.DMA,
        pltpu.SemaphoreType.DMA,
    ],
)


def kernel(x, left_idx, right_idx):
    xv = jnp.transpose(x, (1, 0, 2)).reshape(IN_SZ, W)
    li = left_idx.astype(jnp.int32)
    ri = right_idx.astype(jnp.int32)
    outv = _upsample(xv, li, ri)
    return jnp.transpose(outv.reshape(OUT_SZ, B, D), (1, 0, 2))


# speedup vs baseline: 11.4839x; 1.0989x over previous
import jax
import jax.numpy as jnp
from jax import lax
from jax.experimental import pallas as pl
from jax.experimental.pallas import tpu as pltpu
from jax.experimental.pallas import tpu_sc as plsc

B = 4
IN_SZ = 40962
OUT_SZ = 163842
D = 128
W = B * D
NEW = OUT_SZ - IN_SZ
NC, NS = 2, 16
NW = NC * NS

GPW = NEW // NW
K = 40
T = GPW // K

IPW = 1280
IK = 32
IC = IPW // IK
ITAIL = IN_SZ - NW * IPW

_mesh = plsc.VectorSubcoreMesh(
    core_axis_name="c", subcore_axis_name="s", num_cores=NC, num_subcores=NS)
_params = pltpu.CompilerParams(use_tc_tiling_on_sc=False)


def _upsample_body(xv_hbm, li_hbm, ri_hbm, out_hbm,
                   idx_l, idx_r, rl0, rr0, rl1, rr1, id0, id1,
                   s_g0, s_g1, s_o0, s_o1, s_ii0, s_ii1, s_io0, s_io1):
    wid = lax.axis_index("s") * NC + lax.axis_index("c")
    rls = (rl0, rl1)
    rrs = (rr0, rr1)
    ids = (id0, id1)
    gsems = (s_g0, s_g1)
    osems = (s_o0, s_o1)
    iisems = (s_ii0, s_ii1)
    iosems = (s_io0, s_io1)

    cl = pltpu.async_copy(li_hbm.at[pl.ds(wid * GPW, GPW)], idx_l, s_g0)
    cr = pltpu.async_copy(ri_hbm.at[pl.ds(wid * GPW, GPW)], idx_r, s_g1)
    cl.wait()
    cr.wait()

    def issue_gather(t, p):
        off = t * K
        pltpu.async_copy(xv_hbm.at[idx_l.at[pl.ds(off, K)]], rls[p], gsems[p])
        pltpu.async_copy(xv_hbm.at[idx_r.at[pl.ds(off, K)]], rrs[p], gsems[p])

    def wait_gather(p):
        pltpu.make_async_copy(xv_hbm.at[pl.ds(0, K)], rls[p], gsems[p]).wait()
        pltpu.make_async_copy(xv_hbm.at[pl.ds(0, K)], rrs[p], gsems[p]).wait()

    def issue_out(t, p):
        pltpu.async_copy(rls[p],
                         out_hbm.at[pl.ds(IN_SZ + wid * GPW + t * K, K)],
                         osems[p])

    def wait_out(p):
        pltpu.make_async_copy(xv_hbm.at[pl.ds(0, K)], rls[p], osems[p]).wait()

    def avg(p):
        rl, rr = rls[p], rrs[p]

        def row(i, carry):
            for j in range(W // 16):
                s = pl.ds(j * 16, 16)
                rl[i, s] = (rl[i, s] + rr[i, s]) * 0.5
            return carry

        lax.fori_loop(0, K, row, 0)

    def step(t, p, first=False, last=None):
        q = 1 - p
        if not first:
            wait_out(q)
        if last is None:
            issue_gather(t + 1, q)
        else:
            pl.when(jnp.logical_not(last))(lambda: issue_gather(t + 1, q))
        wait_gather(p)
        avg(p)
        issue_out(t, p)

    def issue_id_in(c, p):
        pltpu.async_copy(xv_hbm.at[pl.ds(wid * IPW + c * IK, IK)], ids[p],
                         iisems[p])

    def wait_id_in(p):
        pltpu.make_async_copy(xv_hbm.at[pl.ds(0, IK)], ids[p],
                              iisems[p]).wait()

    def issue_id_out(c, p):
        pltpu.async_copy(ids[p], out_hbm.at[pl.ds(wid * IPW + c * IK, IK)],
                         iosems[p])

    def wait_id_out(p):
        pltpu.make_async_copy(xv_hbm.at[pl.ds(0, IK)], ids[p],
                              iosems[p]).wait()

    def id_slot(c, pc, first_slot=False, has_next=None, active=None):
        def body():
            qc = 1 - pc
            if not first_slot:
                wait_id_out(qc)
            if has_next is None:
                issue_id_in(c + 1, qc)
            else:
                pl.when(has_next)(lambda: issue_id_in(c + 1, qc))
            wait_id_in(pc)
            issue_id_out(c, pc)

        if active is None:
            body()
        else:
            pl.when(active)(body)

    issue_gather(0, 0)
    issue_id_in(0, 0)
    step(0, 0, first=True)
    id_slot(0, 0, first_slot=True)
    step(1, 1)
    step(2, 0)
    id_slot(1, 1)
    step(3, 1)

    def block(k, carry):
        t0 = 4 * k
        active = k < IC // 2
        step(t0, 0)
        id_slot(2 * k, 0, active=active)
        step(t0 + 1, 1)
        step(t0 + 2, 0)
        id_slot(2 * k + 1, 1, has_next=(k < IC // 2 - 1), active=active)
        step(t0 + 3, 1, last=(k == T // 4 - 1))
        return carry

    lax.fori_loop(1, T // 4, block, 0)

    wait_out(1)
    wait_id_out(1)

    @pl.when(wid == 0)
    def _tails():
        r0 = NW * IPW
        pltpu.sync_copy(xv_hbm.at[pl.ds(r0, ITAIL)], id0.at[pl.ds(0, ITAIL)])
        pltpu.sync_copy(id0.at[pl.ds(0, ITAIL)],
                        out_hbm.at[pl.ds(r0, ITAIL)])

_upsample = pl.kernel(
    _upsample_body,
    out_type=jax.ShapeDtypeStruct((OUT_SZ, W), jnp.float32),
    mesh=_mesh,
    compiler_params=_params,
    scratch_types=[
        pltpu.VMEM((GPW,), jnp.int32),
        pltpu.VMEM((GPW,), jnp.int32),
        pltpu.VMEM((K, W), jnp.float32),
        pltpu.VMEM((K, W), jnp.float32),
        pltpu.VMEM((K, W), jnp.float32),
        pltpu.VMEM((K, W), jnp.float32),
        pltpu.VMEM((IK, W), jnp.float32),
        pltpu.VMEM((IK, W), jnp.float32),
        pltpu.SemaphoreType.DMA,
        pltpu.SemaphoreType.DMA,
        pltpu.SemaphoreType.DMA,
        pltpu.SemaphoreType.DMA,
        pltpu.SemaphoreType.DMA,
        pltpu.SemaphoreType.DMA,
        pltpu.SemaphoreType.DMA,
        pltpu.SemaphoreType.DMA,
    ],
)


def kernel(x, left_idx, right_idx):
    xv = jnp.transpose(x, (1, 0, 2)).reshape(IN_SZ, W)
    li = left_idx[IN_SZ:].astype(jnp.int32)
    ri = right_idx[IN_SZ:].astype(jnp.int32)
    outv = _upsample(xv, li, ri)
    return jnp.transpose(outv.reshape(OUT_SZ, B, D), (1, 0, 2))


# speedup vs baseline: 11.5286x; 1.0039x over previous
import jax
import jax.numpy as jnp
from jax import lax
from jax.experimental import pallas as pl
from jax.experimental.pallas import tpu as pltpu
from jax.experimental.pallas import tpu_sc as plsc

B = 4
IN_SZ = 40962
OUT_SZ = 163842
D = 128
W = B * D
NEW = OUT_SZ - IN_SZ
NC, NS = 2, 16
NW = NC * NS

GPW = NEW // NW
K = 48
T = GPW // K

IPW = 1280
IK = 16
IC = IPW // IK
ITAIL = IN_SZ - NW * IPW

_mesh = plsc.VectorSubcoreMesh(
    core_axis_name="c", subcore_axis_name="s", num_cores=NC, num_subcores=NS)
_params = pltpu.CompilerParams(use_tc_tiling_on_sc=False)


def _upsample_body(xv_hbm, li_hbm, ri_hbm, out_hbm,
                   idx_l, idx_r, rl0, rr0, rl1, rr1, id0, id1,
                   s_g0, s_g1, s_o0, s_o1, s_ii0, s_ii1, s_io0, s_io1):
    wid = lax.axis_index("s") * NC + lax.axis_index("c")
    rls = (rl0, rl1)
    rrs = (rr0, rr1)
    ids = (id0, id1)
    gsems = (s_g0, s_g1)
    osems = (s_o0, s_o1)
    iisems = (s_ii0, s_ii1)
    iosems = (s_io0, s_io1)

    cl = pltpu.async_copy(li_hbm.at[pl.ds(wid * GPW, GPW)], idx_l, s_g0)
    cr = pltpu.async_copy(ri_hbm.at[pl.ds(wid * GPW, GPW)], idx_r, s_g1)
    cl.wait()
    cr.wait()

    def issue_gather(t, p):
        off = t * K
        pltpu.async_copy(xv_hbm.at[idx_l.at[pl.ds(off, K)]], rls[p], gsems[p])
        pltpu.async_copy(xv_hbm.at[idx_r.at[pl.ds(off, K)]], rrs[p], gsems[p])

    def wait_gather(p):
        pltpu.make_async_copy(xv_hbm.at[pl.ds(0, K)], rls[p], gsems[p]).wait()
        pltpu.make_async_copy(xv_hbm.at[pl.ds(0, K)], rrs[p], gsems[p]).wait()

    def issue_out(t, p):
        pltpu.async_copy(rls[p],
                         out_hbm.at[pl.ds(IN_SZ + wid * GPW + t * K, K)],
                         osems[p])

    def wait_out(p):
        pltpu.make_async_copy(xv_hbm.at[pl.ds(0, K)], rls[p], osems[p]).wait()

    def avg(p):
        rl, rr = rls[p], rrs[p]

        def row(i, carry):
            for j in range(W // 16):
                s = pl.ds(j * 16, 16)
                rl[i, s] = (rl[i, s] + rr[i, s]) * 0.5
            return carry

        lax.fori_loop(0, K, row, 0)

    def issue_id_in(c, p):
        pltpu.async_copy(xv_hbm.at[pl.ds(wid * IPW + c * IK, IK)], ids[p],
                         iisems[p])

    def wait_id_in(p):
        pltpu.make_async_copy(xv_hbm.at[pl.ds(0, IK)], ids[p],
                              iisems[p]).wait()

    def issue_id_out(c, p):
        pltpu.async_copy(ids[p], out_hbm.at[pl.ds(wid * IPW + c * IK, IK)],
                         iosems[p])

    def wait_id_out(p):
        pltpu.make_async_copy(xv_hbm.at[pl.ds(0, IK)], ids[p],
                              iosems[p]).wait()

    def step(t, p, first=False, last=None):
        q = 1 - p
        if not first:
            wait_out(q)
            wait_id_out(q)
        if last is None:
            issue_gather(t + 1, q)
            issue_id_in(t + 1, q)
        else:
            def _issue_next():
                issue_gather(t + 1, q)
                issue_id_in(t + 1, q)

            pl.when(jnp.logical_not(last))(_issue_next)
        wait_gather(p)
        avg(p)
        issue_out(t, p)
        wait_id_in(p)
        issue_id_out(t, p)

    issue_gather(0, 0)
    issue_id_in(0, 0)
    step(0, 0, first=True)
    step(1, 1)

    def two_steps(k, carry):
        t0 = 2 * k
        step(t0, 0)
        step(t0 + 1, 1, last=(k == T // 2 - 1))
        return carry

    lax.fori_loop(1, T // 2, two_steps, 0)

    wait_out(1)
    wait_id_out(1)

    @pl.when(wid == 0)
    def _tails():
        r0 = NW * IPW
        pltpu.sync_copy(xv_hbm.at[pl.ds(r0, ITAIL)], id0.at[pl.ds(0, ITAIL)])
        pltpu.sync_copy(id0.at[pl.ds(0, ITAIL)],
                        out_hbm.at[pl.ds(r0, ITAIL)])

_upsample = pl.kernel(
    _upsample_body,
    out_type=jax.ShapeDtypeStruct((OUT_SZ, W), jnp.float32),
    mesh=_mesh,
    compiler_params=_params,
    scratch_types=[
        pltpu.VMEM((GPW,), jnp.int32),
        pltpu.VMEM((GPW,), jnp.int32),
        pltpu.VMEM((K, W), jnp.float32),
        pltpu.VMEM((K, W), jnp.float32),
        pltpu.VMEM((K, W), jnp.float32),
        pltpu.VMEM((K, W), jnp.float32),
        pltpu.VMEM((IK, W), jnp.float32),
        pltpu.VMEM((IK, W), jnp.float32),
        pltpu.SemaphoreType.DMA,
        pltpu.SemaphoreType.DMA,
        pltpu.SemaphoreType.DMA,
        pltpu.SemaphoreType.DMA,
        pltpu.SemaphoreType.DMA,
        pltpu.SemaphoreType.DMA,
        pltpu.SemaphoreType.DMA,
        pltpu.SemaphoreType.DMA,
    ],
)


def kernel(x, left_idx, right_idx):
    xv = jnp.transpose(x, (1, 0, 2)).reshape(IN_SZ, W)
    li = left_idx[IN_SZ:].astype(jnp.int32)
    ri = right_idx[IN_SZ:].astype(jnp.int32)
    outv = _upsample(xv, li, ri)
    return jnp.transpose(outv.reshape(OUT_SZ, B, D), (1, 0, 2))
